# baseline (speedup 1.0000x reference)
import jax
import jax.numpy as jnp
from jax import lax
from jax.experimental import pallas as pl
from jax.experimental.pallas import tpu as pltpu

B, R, N, D = 4, 4, 1024, 256
NTILE = 512
NT = N // NTILE
L = 2


def _body(adj_ref, x_ref, xown_ref, wrt_ref, br_ref, w0_ref, b0_ref,
          out0_ref, out1_ref,
          acache_ref, x1_ref, den_ref, ycat_ref, agg_ref, dacc_ref):
    l = pl.program_id(0)
    b = pl.program_id(1)
    n = pl.program_id(2)
    r = pl.program_id(3)
    bn = b * NT + n

    @pl.when(n == 0)
    def _ycat():
        xs = lax.cond(l == 0, lambda: x_ref[0], lambda: x1_ref[b])
        yc = lax.dot_general(xs, wrt_ref[0, 0], (((1,), (0,)), ((), ())),
                             preferred_element_type=jnp.float32)
        yc = yc + br_ref[0, 0]
        for k in range(R):
            @pl.when(r == k)
            def _():
                ycat_ref[k * N:(k + 1) * N, :] = yc.astype(jnp.bfloat16)

    @pl.when(l == 0)
    def _layer0():
        adj_blk = adj_ref[0, 0]
        rowsum = jnp.sum(adj_blk, axis=1, keepdims=True)
        adj_bf = adj_blk.astype(jnp.bfloat16)
        for k in range(R):
            @pl.when(r == k)
            def _():
                acache_ref[bn, :, k * N:(k + 1) * N] = adj_bf
        contrib = jnp.dot(adj_bf, ycat_ref[pl.ds(r * N, N), :],
                          preferred_element_type=jnp.float32)

        @pl.when(r == 0)
        def _():
            agg_ref[...] = contrib
            dacc_ref[...] = rowsum

        @pl.when(r > 0)
        def _():
            agg_ref[...] += contrib
            dacc_ref[...] += rowsum

        @pl.when(r == R - 1)
        def _():
            den_ref[bn] = dacc_ref[...] + 1.0

    @pl.when((l == 1) & (r == R - 1))
    def _layer1():
        agg_ref[...] = jnp.dot(acache_ref[bn], ycat_ref[...],
                               preferred_element_type=jnp.float32)

    @pl.when(r == R - 1)
    def _finish():
        x_own = lax.cond(l == 0,
                         lambda: xown_ref[0],
                         lambda: x1_ref[b, pl.ds(n * NTILE, NTILE)])
        h0 = lax.dot_general(x_own, w0_ref[0], (((1,), (1,)), ((), ())),
                             preferred_element_type=jnp.float32)
        out = jnp.maximum((agg_ref[...] + h0 + b0_ref[0]) / den_ref[bn], 0.0)

        @pl.when(l == 0)
        def _():
            out0_ref[0] = out
            x1_ref[b, pl.ds(n * NTILE, NTILE)] = out.astype(jnp.bfloat16)

        @pl.when(l == 1)
        def _():
            out1_ref[0] = out


@jax.jit
def kernel(nodes, adj, W0, b0, Wr, br):
    bf = jnp.bfloat16
    xbf = nodes.astype(bf)
    wrt = Wr.transpose(0, 1, 3, 2).astype(bf)

    out0, out1 = pl.pallas_call(
        _body,
        grid=(L, B, NT, R),
        in_specs=[
            pl.BlockSpec((1, 1, NTILE, N),
                         lambda l, b, n, r: (jnp.where(l == 0, b, 0),
                                             jnp.where(l == 0, r, 0),
                                             jnp.where(l == 0, n, 0), 0)),
            pl.BlockSpec((1, N, D),
                         lambda l, b, n, r: (jnp.where(l == 0, b, 0), 0, 0)),
            pl.BlockSpec((1, NTILE, D),
                         lambda l, b, n, r: (jnp.where(l == 0, b, 0),
                                             jnp.where(l == 0, n, 0), 0)),
            pl.BlockSpec((1, 1, D, D), lambda l, b, n, r: (l, r, 0, 0)),
            pl.BlockSpec((1, 1, 1, D), lambda l, b, n, r: (l, r, 0, 0)),
            pl.BlockSpec((1, D, D), lambda l, b, n, r: (l, 0, 0)),
            pl.BlockSpec((1, 1, D), lambda l, b, n, r: (l, 0, 0)),
        ],
        out_specs=[
            pl.BlockSpec((1, NTILE, D),
                         lambda l, b, n, r: (jnp.where(l == 0, b, B - 1),
                                             jnp.where(l == 0, n, NT - 1), 0)),
            pl.BlockSpec((1, NTILE, D),
                         lambda l, b, n, r: (jnp.where(l == 0, 0, b),
                                             jnp.where(l == 0, 0, n), 0)),
        ],
        out_shape=[
            jax.ShapeDtypeStruct((B, N, D), jnp.float32),
            jax.ShapeDtypeStruct((B, N, D), jnp.float32),
        ],
        scratch_shapes=[
            pltpu.VMEM((B * NT, NTILE, R * N), jnp.bfloat16),
            pltpu.VMEM((B, N, D), jnp.bfloat16),
            pltpu.VMEM((B * NT, NTILE, 1), jnp.float32),
            pltpu.VMEM((R * N, D), jnp.bfloat16),
            pltpu.VMEM((NTILE, D), jnp.float32),
            pltpu.VMEM((NTILE, 1), jnp.float32),
        ],
    )(adj, xbf, xbf, wrt, br[:, :, None, :], W0.astype(bf), b0[:, None, :])
    return (out0, out1)


# speedup vs baseline: 1.0592x; 1.0592x over previous
import jax
import jax.numpy as jnp
from jax import lax
from jax.experimental import pallas as pl
from jax.experimental.pallas import tpu as pltpu

B, R, N, D = 4, 4, 1024, 256
NTILE = 512
NT = N // NTILE
L = 2


def _stage_s(r, adj_bf, x_full, scat_ref):
    s = jnp.dot(adj_bf, x_full, preferred_element_type=jnp.float32)
    sbf = s.astype(jnp.bfloat16)
    for k in range(R):
        @pl.when(r == k)
        def _():
            scat_ref[:, k * D:(k + 1) * D] = sbf


def _body(adj_ref, x_ref, xown_ref, wcat_ref, brm_ref, w0_ref, b0_ref,
          out0_ref, out1_ref,
          acache_ref, x1_ref, bias1_ref, den_ref,
          scat_ref, rsm_ref, dacc_ref):
    l = pl.program_id(0)
    b = pl.program_id(1)
    n = pl.program_id(2)
    r = pl.program_id(3)
    idx = (b * NT + n) * R + r
    bn = b * NT + n

    @pl.when(l == 0)
    def _layer0():
        adj_blk = adj_ref[0, 0]
        rowsum = jnp.sum(adj_blk, axis=1, keepdims=True)
        adj_bf = adj_blk.astype(jnp.bfloat16)
        acache_ref[idx] = adj_bf

        @pl.when(r == 0)
        def _():
            rsm_ref[...] = jnp.zeros((NTILE, 128), jnp.float32)
            dacc_ref[...] = rowsum

        for k in range(R):
            @pl.when(r == k)
            def _():
                rsm_ref[:, k:k + 1] = rowsum

        @pl.when(r > 0)
        def _():
            dacc_ref[...] += rowsum

        _stage_s(r, adj_bf, x_ref[0], scat_ref)

    @pl.when(l == 1)
    def _layer1():
        _stage_s(r, acache_ref[idx], x1_ref[b], scat_ref)

    @pl.when(r == R - 1)
    def _finish():
        agg = jnp.dot(scat_ref[...], wcat_ref[0],
                      preferred_element_type=jnp.float32)

        @pl.when(l == 0)
        def _():
            den_ref[bn] = dacc_ref[...] + 1.0
            rsm = rsm_ref[...]
            bias1_ref[bn] = jnp.dot(rsm, brm_ref[1],
                                    preferred_element_type=jnp.float32)

        bias = lax.cond(
            l == 0,
            lambda: jnp.dot(rsm_ref[...], brm_ref[0],
                            preferred_element_type=jnp.float32),
            lambda: bias1_ref[bn])

        x_own = lax.cond(l == 0,
                         lambda: xown_ref[0],
                         lambda: x1_ref[b, pl.ds(n * NTILE, NTILE)])
        h0 = lax.dot_general(x_own, w0_ref[0], (((1,), (1,)), ((), ())),
                             preferred_element_type=jnp.float32)
        out = jnp.maximum((agg + bias + h0 + b0_ref[0]) / den_ref[bn], 0.0)

        @pl.when(l == 0)
        def _():
            out0_ref[0] = out
            x1_ref[b, pl.ds(n * NTILE, NTILE)] = out.astype(jnp.bfloat16)

        @pl.when(l == 1)
        def _():
            out1_ref[0] = out


@jax.jit
def kernel(nodes, adj, W0, b0, Wr, br):
    bf = jnp.bfloat16
    xbf = nodes.astype(bf)
    wcat = Wr.transpose(0, 1, 3, 2).reshape(L, R * D, D).astype(bf)
    brm = jnp.zeros((L, 128, D), jnp.float32).at[:, :R, :].set(br)

    out0, out1 = pl.pallas_call(
        _body,
        grid=(L, B, NT, R),
        in_specs=[
            pl.BlockSpec((1, 1, NTILE, N),
                         lambda l, b, n, r: (jnp.where(l == 0, b, 0),
                                             jnp.where(l == 0, r, 0),
                                             jnp.where(l == 0, n, 0), 0)),
            pl.BlockSpec((1, N, D),
                         lambda l, b, n, r: (jnp.where(l == 0, b, 0), 0, 0)),
            pl.BlockSpec((1, NTILE, D),
                         lambda l, b, n, r: (jnp.where(l == 0, b, 0),
                                             jnp.where(l == 0, n, 0), 0)),
            pl.BlockSpec((1, R * D, D), lambda l, b, n, r: (l, 0, 0)),
            pl.BlockSpec((L, 128, D), lambda l, b, n, r: (0, 0, 0)),
            pl.BlockSpec((1, D, D), lambda l, b, n, r: (l, 0, 0)),
            pl.BlockSpec((1, 1, D), lambda l, b, n, r: (l, 0, 0)),
        ],
        out_specs=[
            pl.BlockSpec((1, NTILE, D),
                         lambda l, b, n, r: (jnp.where(l == 0, b, B - 1),
                                             jnp.where(l == 0, n, NT - 1), 0)),
            pl.BlockSpec((1, NTILE, D),
                         lambda l, b, n, r: (jnp.where(l == 0, 0, b),
                                             jnp.where(l == 0, 0, n), 0)),
        ],
        out_shape=[
            jax.ShapeDtypeStruct((B, N, D), jnp.float32),
            jax.ShapeDtypeStruct((B, N, D), jnp.float32),
        ],
        scratch_shapes=[
            pltpu.VMEM((B * NT * R, NTILE, N), jnp.bfloat16),
            pltpu.VMEM((B, N, D), jnp.bfloat16),
            pltpu.VMEM((B * NT, NTILE, D), jnp.float32),
            pltpu.VMEM((B * NT, NTILE, 1), jnp.float32),
            pltpu.VMEM((NTILE, R * D), jnp.bfloat16),
            pltpu.VMEM((NTILE, 128), jnp.float32),
            pltpu.VMEM((NTILE, 1), jnp.float32),
        ],
    )(adj, xbf, xbf, wcat, brm, W0.astype(bf), b0[:, None, :])
    return (out0, out1)


# speedup vs baseline: 1.2952x; 1.2228x over previous
import jax
import jax.numpy as jnp
from jax import lax
from jax.experimental import pallas as pl
from jax.experimental.pallas import tpu as pltpu

B, R, N, D = 4, 4, 1024, 256
NTILE = 512
NT = N // NTILE
L = 2


def _finish_tile(scat_ref, wcat_ref, wl, bias, x_own, w0_ref, b0_ref, den):
    agg = jnp.dot(scat_ref[...], wcat_ref[wl, 0],
                  preferred_element_type=jnp.float32)
    h0 = lax.dot_general(x_own, w0_ref[wl, 0], (((1,), (1,)), ((), ())),
                         preferred_element_type=jnp.float32)
    return jnp.maximum((agg + bias + h0 + b0_ref[wl, 0]) / den, 0.0)


def _body(adj_ref, x_ref, xown_ref, wcat_ref, brm_ref, w0_ref, b0_ref,
          out0_ref, out1_ref,
          acache_ref, x1_ref, bias1_ref, den_ref,
          scat0_ref, scat1_ref, rsm_ref, dacc_ref):
    bb = pl.program_id(0)
    n = pl.program_id(1)
    r = pl.program_id(2)

    @pl.when(bb < B)
    def _layer0():
        bn = bb * NT + n
        idx = bn * R + r
        adj_blk = adj_ref[0, 0]
        rowsum = jnp.sum(adj_blk, axis=1, keepdims=True)
        adj_bf = adj_blk.astype(jnp.bfloat16)
        acache_ref[idx] = adj_bf

        @pl.when(r == 0)
        def _():
            rsm_ref[...] = jnp.zeros((NTILE, 128), jnp.float32)
            dacc_ref[...] = rowsum

        for k in range(R):
            @pl.when(r == k)
            def _():
                rsm_ref[:, k:k + 1] = rowsum

        @pl.when(r > 0)
        def _():
            dacc_ref[...] += rowsum

        s = jnp.dot(adj_bf, x_ref[0], preferred_element_type=jnp.float32)
        sbf = s.astype(jnp.bfloat16)
        for k in range(R):
            @pl.when(r == k)
            def _():
                scat0_ref[:, k * D:(k + 1) * D] = sbf

        @pl.when(r == R - 1)
        def _():
            den = dacc_ref[...] + 1.0
            den_ref[bn] = den
            rsm = rsm_ref[...]
            bias1_ref[bn] = jnp.dot(rsm, brm_ref[1, 0],
                                    preferred_element_type=jnp.float32)
            bias0 = jnp.dot(rsm, brm_ref[0, 0],
                            preferred_element_type=jnp.float32)
            out = _finish_tile(scat0_ref, wcat_ref, 0, bias0,
                               xown_ref[0], w0_ref, b0_ref, den)
            out0_ref[0] = out
            x1_ref[bb, pl.ds(n * NTILE, NTILE)] = out.astype(jnp.bfloat16)

    @pl.when(bb >= 1)
    def _layer1():
        bp = bb - 1
        bn = bp * NT + n
        idx = bn * R + r
        s = jnp.dot(acache_ref[idx], x1_ref[bp],
                    preferred_element_type=jnp.float32)
        sbf = s.astype(jnp.bfloat16)
        for k in range(R):
            @pl.when(r == k)
            def _():
                scat1_ref[:, k * D:(k + 1) * D] = sbf

        @pl.when(r == R - 1)
        def _():
            out = _finish_tile(scat1_ref, wcat_ref, 1, bias1_ref[bn],
                               x1_ref[bp, pl.ds(n * NTILE, NTILE)],
                               w0_ref, b0_ref, den_ref[bn])
            out1_ref[0] = out


@jax.jit
def kernel(nodes, adj, W0, b0, Wr, br):
    bf = jnp.bfloat16
    xbf = nodes.astype(bf)
    wcat = Wr.transpose(0, 1, 3, 2).reshape(L, 1, R * D, D).astype(bf)
    brm = jnp.zeros((L, 1, 128, D), jnp.float32).at[:, 0, :R, :].set(br)

    out0, out1 = pl.pallas_call(
        _body,
        grid=(B + 1, NT, R),
        in_specs=[
            pl.BlockSpec((1, 1, NTILE, N),
                         lambda bb, n, r: (jnp.minimum(bb, B - 1),
                                           jnp.where(bb < B, r, 0),
                                           jnp.where(bb < B, n, 0), 0)),
            pl.BlockSpec((1, N, D),
                         lambda bb, n, r: (jnp.minimum(bb, B - 1), 0, 0)),
            pl.BlockSpec((1, NTILE, D),
                         lambda bb, n, r: (jnp.minimum(bb, B - 1),
                                           jnp.where(bb < B, n, 0), 0)),
            pl.BlockSpec((L, 1, R * D, D), lambda bb, n, r: (0, 0, 0, 0)),
            pl.BlockSpec((L, 1, 128, D), lambda bb, n, r: (0, 0, 0, 0)),
            pl.BlockSpec((L, 1, D, D), lambda bb, n, r: (0, 0, 0, 0)),
            pl.BlockSpec((L, 1, 1, D), lambda bb, n, r: (0, 0, 0, 0)),
        ],
        out_specs=[
            pl.BlockSpec((1, NTILE, D),
                         lambda bb, n, r: (jnp.minimum(bb, B - 1),
                                           jnp.where(bb < B, n, NT - 1), 0)),
            pl.BlockSpec((1, NTILE, D),
                         lambda bb, n, r: (jnp.maximum(bb - 1, 0),
                                           jnp.where(bb >= 1, n, 0), 0)),
        ],
        out_shape=[
            jax.ShapeDtypeStruct((B, N, D), jnp.float32),
            jax.ShapeDtypeStruct((B, N, D), jnp.float32),
        ],
        scratch_shapes=[
            pltpu.VMEM((B * NT * R, NTILE, N), jnp.bfloat16),
            pltpu.VMEM((B, N, D), jnp.bfloat16),
            pltpu.VMEM((B * NT, NTILE, D), jnp.float32),
            pltpu.VMEM((B * NT, NTILE, 1), jnp.float32),
            pltpu.VMEM((NTILE, R * D), jnp.bfloat16),
            pltpu.VMEM((NTILE, R * D), jnp.bfloat16),
            pltpu.VMEM((NTILE, 128), jnp.float32),
            pltpu.VMEM((NTILE, 1), jnp.float32),
        ],
    )(adj, xbf, xbf, wcat, brm, W0[:, None].astype(bf),
      b0[:, None, None, :])
    return (out0, out1)
